# baseline (speedup 1.0000x reference)
import functools

import jax
import jax.numpy as jnp
import numpy as np
from jax import lax
from jax.experimental import pallas as pl
from jax.experimental.pallas import tpu as pltpu
from jax.experimental.pallas import tpu_sc as plsc

H = 256
HEADS = 8
DK = 32
N_REAL = 10000
NP = 10240
NE = 160000
UW = 384
ZW = 512
AC = 128
OUTP = 128

BN = 512
BE = 800
GB = 200
SB = 200

_f32 = jnp.float32
_bf16 = jnp.bfloat16


def _bdot(a, w):
    return jnp.dot(a.astype(_bf16), w, preferred_element_type=_f32)


def _full(shape):
    return pl.BlockSpec(shape, lambda i: (0,) * len(shape))


def _rows(width, blk):
    return pl.BlockSpec((blk, width), lambda i: (i, 0))


def _ln(x, g, b):
    mu = jnp.mean(x, axis=-1, keepdims=True)
    var = jnp.mean((x - mu) ** 2, axis=-1, keepdims=True)
    return (x - mu) * lax.rsqrt(var + 1e-5) * g + b


def _pack_kv(k, v):
    ku = lax.bitcast_convert_type(k.astype(_bf16), jnp.uint16)
    vu = lax.bitcast_convert_type(v.astype(_bf16), jnp.uint16)
    w = ku.astype(jnp.uint32) | (vu.astype(jnp.uint32) << 16)
    return lax.bitcast_convert_type(w, _f32)


def _unpack_kv(kv):
    w = lax.bitcast_convert_type(kv, jnp.uint32)
    k = lax.bitcast_convert_type(w << 16, _f32)
    v = lax.bitcast_convert_type(w & jnp.uint32(0xFFFF0000), _f32)
    return k, v


def _qkv_body(h_ref, wq, bq, wk, bk, wv, bv,
              qp_o, kv0_o, kv1_o):
    hb = h_ref[...]
    q = jnp.dot(hb, wq[...], preferred_element_type=_f32) + bq[...]
    k = jnp.dot(hb, wk[...], preferred_element_type=_f32) + bk[...]
    v = jnp.dot(hb, wv[...], preferred_element_type=_f32) + bv[...]
    qp_o[...] = _pack_kv(q[:, :AC], q[:, AC:])
    kv0_o[...] = _pack_kv(k[:, :AC], v[:, :AC])
    kv1_o[...] = _pack_kv(k[:, AC:], v[:, AC:])


def _tc_qkv(h, wq, bq, wk, bk, wv320, bv320):
    half = jax.ShapeDtypeStruct((NP, AC), _f32)
    return pl.pallas_call(
        _qkv_body,
        grid=(NP // BN,),
        in_specs=[_rows(H, BN), _full((H, H)), _full((1, H)),
                  _full((H, H)), _full((1, H)),
                  _full((H, H)), _full((1, H))],
        out_specs=[_rows(AC, BN)] * 3,
        out_shape=[half] * 3,
    )(h, wq, bq, wk, bk, wv320, bv320)


def _matbias_body(x_ref, w, b, o_ref):
    y = jnp.dot(x_ref[...], w[...], preferred_element_type=_f32) + b[...]
    o_ref[...] = y.astype(o_ref.dtype)


def _tc_matbias(x, w, b, blk, out_dtype=_f32):
    rows = x.shape[0]
    cols = w.shape[1]
    return pl.pallas_call(
        _matbias_body,
        grid=(rows // blk,),
        in_specs=[_rows(x.shape[1], blk), _full((x.shape[1], cols)),
                  _full((1, cols))],
        out_specs=_rows(cols, blk),
        out_shape=jax.ShapeDtypeStruct((rows, cols), out_dtype),
    )(x, w, b)


def _edge_body(kvs0_ref, kvs1_ref, qps_ref, e_ref,
               ew, eb, ow, ob, f1w, f1b, f2w, f2b, g1, b1, g2, b2,
               u_o, e2_o):
    ks0, vs0 = _unpack_kv(kvs0_ref[...])
    ks1, vs1 = _unpack_kv(kvs1_ref[...])
    ks = jnp.concatenate([ks0, ks1], axis=1)
    q0, q1 = _unpack_kv(qps_ref[...])
    qd = jnp.concatenate([q0, q1], axis=1)
    eblk = e_ref[...]
    ep = _bdot(eblk, ew[...]) + eb[...]
    sarr = ks * qd * ep
    ci = lax.broadcasted_iota(jnp.int32, (H, HEADS), 0) // DK
    hi = lax.broadcasted_iota(jnp.int32, (H, HEADS), 1)
    msum = (ci == hi).astype(_f32)
    att = jnp.exp(jnp.clip(
        jnp.dot(sarr, msum, preferred_element_type=_f32), -5.0, 5.0))
    hb2 = lax.broadcasted_iota(jnp.int32, (HEADS, UW), 0)
    cb2 = lax.broadcasted_iota(jnp.int32, (HEADS, UW), 1)
    mbc = (jnp.where(cb2 < H, cb2 // DK, cb2 - H) == hb2).astype(_f32)
    vsc = jnp.concatenate(
        [vs0, vs1, jnp.ones((qps_ref.shape[0], UW - H), _f32)], axis=1)
    u_o[...] = jnp.dot(att, mbc, preferred_element_type=_f32) * vsc
    e_o = _bdot(sarr, ow[...]) + ob[...]
    e1 = _ln(eblk.astype(_f32) + e_o, g1[...], b1[...])
    ef = _bdot(jnp.maximum(_bdot(e1, f1w[...]) + f1b[...], 0.0),
               f2w[...]) + f2b[...]
    e2_o[...] = _ln(e1 + ef, g2[...], b2[...]).astype(_bf16)


def _tc_edge(kvs0, kvs1, qps, e,
             ew, eb, ow, ob, f1w, f1b, f2w, f2b, g1, b1, g2, b2):
    return pl.pallas_call(
        _edge_body,
        grid=(NE // BE,),
        in_specs=[_rows(AC, BE)] * 3 + [_rows(H, BE),
                  _full((H, H)), _full((1, H)),
                  _full((H, H)), _full((1, H)),
                  _full((H, 2 * H)), _full((1, 2 * H)),
                  _full((2 * H, H)), _full((1, H)),
                  _full((1, H)), _full((1, H)), _full((1, H)), _full((1, H))],
        out_specs=[_rows(UW, BE), _rows(H, BE)],
        out_shape=[jax.ShapeDtypeStruct((NE, UW), _f32),
                   jax.ShapeDtypeStruct((NE, H), _bf16)],
    )(kvs0, kvs1, qps, e,
      ew, eb, ow, ob, f1w, f1b, f2w, f2b, g1, b1, g2, b2)


def _node_body(s_ref, h_ref, ow, ob, f1w, f1b, f2w, f2b, g1, b1, g2, b2, h2_o):
    sblk = s_ref[...]
    wv = sblk[:, :H]
    ri = lax.broadcasted_iota(jnp.int32, (ZW, H), 0)
    ci = lax.broadcasted_iota(jnp.int32, (ZW, H), 1)
    rh = jnp.where(ri >= H + AC, ri - H - AC, ri - H)
    mz = ((ri >= H) & (rh < HEADS) & (ci // DK == rh)).astype(_f32)
    zb = jnp.dot(sblk, mz, preferred_element_type=_f32) + 1e-6
    hat = wv / zb
    h_o = _bdot(hat, ow[...]) + ob[...]
    h1 = _ln(h_ref[...] + h_o, g1[...], b1[...])
    hf = _bdot(jnp.maximum(_bdot(h1, f1w[...]) + f1b[...], 0.0),
               f2w[...]) + f2b[...]
    h2_o[...] = _ln(h1 + hf, g2[...], b2[...])


def _tc_node(sacc, h, ow, ob, f1w, f1b, f2w, f2b, g1, b1, g2, b2):
    return pl.pallas_call(
        _node_body,
        grid=(NP // BN,),
        in_specs=[_rows(ZW, BN), _rows(H, BN),
                  _full((H, H)), _full((1, H)),
                  _full((H, 2 * H)), _full((1, 2 * H)),
                  _full((2 * H, H)), _full((1, H)),
                  _full((1, H)), _full((1, H)), _full((1, H)), _full((1, H))],
        out_specs=_rows(H, BN),
        out_shape=jax.ShapeDtypeStruct((NP, H), _f32),
    )(sacc, h, ow, ob, f1w, f1b, f2w, f2b, g1, b1, g2, b2)


def _sc_gather3(kv0, kv1, qp, src, dst):
    mesh = plsc.VectorSubcoreMesh(core_axis_name="c", subcore_axis_name="s")
    ch = NE // 16
    gb = 160
    np2 = 31
    tb = ch - np2 * 2 * gb
    tr = NP // 16

    @functools.partial(
        pl.kernel,
        out_type=[jax.ShapeDtypeStruct((NE, AC), _f32)] * 3,
        mesh=mesh,
        scratch_types=[
            pltpu.VMEM((gb,), jnp.int32),
            pltpu.VMEM((gb,), jnp.int32),
            pltpu.VMEM((gb, AC), _f32),
            pltpu.VMEM((gb, AC), _f32),
            pltpu.VMEM_SHARED((NP, AC), _f32),
            pltpu.SemaphoreType.DMA,
            pltpu.SemaphoreType.DMA,
        ],
    )
    def kfn(kv0_hbm, kv1_hbm, qp_hbm,
            src_hbm, dst_hbm,
            kvs0_hbm, kvs1_hbm, qps_hbm,
            idx0, idx1, rows0, rows1, spm, semg, semw):
        c = lax.axis_index("c")
        s = lax.axis_index("s")

        def phase(tab_hbm, idx_hbm, out_hbm, base0, ch_t, np_t, tails):
            pltpu.sync_copy(tab_hbm.at[pl.ds(s * tr, tr)],
                            spm.at[pl.ds(s * tr, tr)])
            plsc.subcore_barrier()

            def body(j2, carry):
                base = base0 + s * ch_t + j2 * (2 * gb)

                @pl.when(j2 > 0)
                def _():
                    pltpu.make_async_copy(
                        rows0, out_hbm.at[pl.ds(0, gb)], semw).wait()
                    pltpu.make_async_copy(
                        rows1, out_hbm.at[pl.ds(0, gb)], semw).wait()

                pltpu.sync_copy(idx_hbm.at[pl.ds(base, gb)], idx0)
                pltpu.sync_copy(idx_hbm.at[pl.ds(base + gb, gb)], idx1)
                pltpu.async_copy(spm.at[idx0], rows0, semg).wait()
                pltpu.async_copy(rows0, out_hbm.at[pl.ds(base, gb)], semw)
                pltpu.async_copy(spm.at[idx1], rows1, semg).wait()
                pltpu.async_copy(rows1, out_hbm.at[pl.ds(base + gb, gb)],
                                 semw)
                return carry

            lax.fori_loop(0, np_t, body, 0)
            pltpu.make_async_copy(rows0, out_hbm.at[pl.ds(0, gb)], semw).wait()
            pltpu.make_async_copy(rows1, out_hbm.at[pl.ds(0, gb)], semw).wait()
            tbase = base0 + s * ch_t + np_t * 2 * gb
            for t in tails:
                pltpu.sync_copy(idx_hbm.at[pl.ds(tbase, t)],
                                idx0.at[pl.ds(0, t)])
                pltpu.async_copy(spm.at[idx0.at[pl.ds(0, t)]],
                                 rows0.at[pl.ds(0, t)], semg).wait()
                pltpu.sync_copy(rows0.at[pl.ds(0, t)],
                                out_hbm.at[pl.ds(tbase, t)])
                tbase = tbase + t
            plsc.subcore_barrier()

        @pl.when(c == 0)
        def _():
            phase(kv0_hbm, src_hbm, kvs0_hbm, 0, ch, np2, [tb])
            phase(qp_hbm, dst_hbm, qps_hbm, 0, NE // 32, 15, [160, 40])

        @pl.when(c == 1)
        def _():
            phase(kv1_hbm, src_hbm, kvs1_hbm, 0, ch, np2, [tb])
            phase(qp_hbm, dst_hbm, qps_hbm, NE // 2, NE // 32, 15, [160, 40])

    return kfn(kv0, kv1, qp, src, dst)


def _sc_scatter(u, dst, zinit):
    mesh = plsc.VectorSubcoreMesh(core_axis_name="c", subcore_axis_name="s")
    sb = 128
    ch1 = 9984
    np1 = ch1 // (2 * sb)
    ch2 = 4992
    n2 = ch2 // sb
    zr = NP // 16

    @functools.partial(
        pl.kernel,
        out_type=jax.ShapeDtypeStruct((NP, ZW), _f32),
        mesh=mesh,
        scratch_types=[
            pltpu.VMEM((sb,), jnp.int32),
            pltpu.VMEM((sb,), jnp.int32),
            pltpu.VMEM((sb, AC), _f32),
            pltpu.VMEM((sb, AC), _f32),
            pltpu.VMEM_SHARED((NP, AC), _f32),
            pltpu.SemaphoreType.DMA,
            pltpu.SemaphoreType.DMA,
            pltpu.SemaphoreType.DMA,
            pltpu.SemaphoreType.DMA,
        ],
    )
    def kfn(u_hbm, dst_hbm, z_hbm, out_hbm, idx0, idx1, st0, st1,
            acc, semi0, semu0, semi1, semu1):
        c = lax.axis_index("c")
        s = lax.axis_index("s")
        pltpu.sync_copy(z_hbm, acc.at[pl.ds(s * zr, zr)])
        plsc.subcore_barrier()

        def load(base, idxv, stv, col0, semi, semu):
            pltpu.async_copy(dst_hbm.at[pl.ds(base, sb)], idxv, semi)
            pltpu.async_copy(u_hbm.at[pl.ds(base, sb), pl.ds(col0, AC)],
                             stv, semu)

        def drain(idxv, stv, col0, semi, semu):
            pltpu.make_async_copy(dst_hbm.at[pl.ds(0, sb)], idxv, semi).wait()
            pltpu.make_async_copy(u_hbm.at[pl.ds(0, sb), pl.ds(col0, AC)],
                                  stv, semu).wait()

        def block(base, col0):
            pltpu.sync_copy(dst_hbm.at[pl.ds(base, sb)], idx0)
            pltpu.sync_copy(u_hbm.at[pl.ds(base, sb), pl.ds(col0, AC)], st0)
            pltpu.sync_copy(st0, acc.at[idx0], add=True)

        def sweep1(col0):
            base0 = s * ch1
            load(base0, idx0, st0, col0, semi0, semu0)

            def body(j2, carry):
                base = base0 + j2 * (2 * sb)
                load(base + sb, idx1, st1, col0, semi1, semu1)
                drain(idx0, st0, col0, semi0, semu0)
                pltpu.sync_copy(st0, acc.at[idx0], add=True)

                @pl.when(j2 < np1 - 1)
                def _():
                    load(base + 2 * sb, idx0, st0, col0, semi0, semu0)

                drain(idx1, st1, col0, semi1, semu1)
                pltpu.sync_copy(st1, acc.at[idx1], add=True)
                return carry

            lax.fori_loop(0, np1, body, 0)

            @pl.when(s == 15)
            def _():
                block(16 * ch1, col0)
                block(16 * ch1 + sb, col0)

        def sweep2(col0):
            base0 = c * (NE // 2) + s * ch2

            def body(j, carry):
                block(base0 + j * sb, col0)
                return carry

            lax.fori_loop(0, n2, body, 0)

            @pl.when(s == 15)
            def _():
                block(c * (NE // 2) + 16 * ch2, col0)

        def copyout(col0):
            pltpu.sync_copy(acc.at[pl.ds(s * zr, zr)],
                            out_hbm.at[pl.ds(s * zr, zr), pl.ds(col0, AC)])

        @pl.when(c == 0)
        def _():
            sweep1(0)

        @pl.when(c == 1)
        def _():
            sweep1(AC)

        plsc.subcore_barrier()

        @pl.when(c == 0)
        def _():
            copyout(0)

        @pl.when(c == 1)
        def _():
            copyout(AC)

        pltpu.sync_copy(z_hbm, acc.at[pl.ds(s * zr, zr)])
        plsc.subcore_barrier()

        sweep2(2 * AC)
        plsc.subcore_barrier()

        @pl.when(c == 0)
        def _():
            copyout(2 * AC)

        @pl.when(c == 1)
        def _():
            copyout(3 * AC)

    return kfn(u, dst, zinit)


def kernel(g, h, e, params):
    src = g[0].astype(jnp.int32)
    dst = g[1].astype(jnp.int32)
    h = jnp.pad(h, ((0, NP - h.shape[0]), (0, 0)))
    zinit = jnp.zeros((NP // 16, AC), _f32)
    scale = np.float32(1.0 / np.sqrt(DK))

    def r(b):
        return b.reshape(1, -1)

    for p in params["layers"]:
        qp, kv0, kv1 = _tc_qkv(h, p["Q"]["W"], r(p["Q"]["b"]),
                               p["K"]["W"], r(p["K"]["b"]),
                               p["V"]["W"], r(p["V"]["b"]))
        kvs0, kvs1, qps = _sc_gather3(kv0, kv1, qp, src, dst)
        u, e = _tc_edge(kvs0, kvs1, qps, e,
                        (p["E"]["W"] * scale).astype(_bf16),
                        r(p["E"]["b"]) * scale,
                        p["Oe"]["W"].astype(_bf16), r(p["Oe"]["b"]),
                        p["Fe1"]["W"].astype(_bf16), r(p["Fe1"]["b"]),
                        p["Fe2"]["W"].astype(_bf16), r(p["Fe2"]["b"]),
                        r(p["ln1e_g"]), r(p["ln1e_b"]),
                        r(p["ln2e_g"]), r(p["ln2e_b"]))
        sacc = _sc_scatter(u, dst, zinit)
        h = _tc_node(sacc, h,
                     p["Oh"]["W"].astype(_bf16), r(p["Oh"]["b"]),
                     p["Fh1"]["W"].astype(_bf16), r(p["Fh1"]["b"]),
                     p["Fh2"]["W"].astype(_bf16), r(p["Fh2"]["b"]),
                     r(p["ln1h_g"]), r(p["ln1h_b"]),
                     r(p["ln2h_g"]), r(p["ln2h_b"]))

    cw = jnp.pad(params["cls"]["W"], ((0, 0), (0, OUTP - 40)))
    cb = jnp.pad(params["cls"]["b"], ((0, OUTP - 40),)).reshape(1, OUTP)
    logits = _tc_matbias(h, cw, cb, BN)
    return logits[:N_REAL, :40]


# speedup vs baseline: 1.0105x; 1.0105x over previous
import functools

import jax
import jax.numpy as jnp
import numpy as np
from jax import lax
from jax.experimental import pallas as pl
from jax.experimental.pallas import tpu as pltpu
from jax.experimental.pallas import tpu_sc as plsc

H = 256
HEADS = 8
DK = 32
N_REAL = 10000
NP = 10240
NE = 160000
UW = 384
ZW = 512
AC = 128
OUTP = 128

BN = 512
BE = 800
GB = 200
SB = 200

_f32 = jnp.float32
_bf16 = jnp.bfloat16


def _bdot(a, w):
    return jnp.dot(a.astype(_bf16), w, preferred_element_type=_f32)


def _full(shape):
    return pl.BlockSpec(shape, lambda i: (0,) * len(shape))


def _rows(width, blk):
    return pl.BlockSpec((blk, width), lambda i: (i, 0))


def _ln(x, g, b):
    mu = jnp.mean(x, axis=-1, keepdims=True)
    var = jnp.mean((x - mu) ** 2, axis=-1, keepdims=True)
    return (x - mu) * lax.rsqrt(var + 1e-5) * g + b


def _pack_kv(k, v):
    ku = lax.bitcast_convert_type(k.astype(_bf16), jnp.uint16)
    vu = lax.bitcast_convert_type(v.astype(_bf16), jnp.uint16)
    w = ku.astype(jnp.uint32) | (vu.astype(jnp.uint32) << 16)
    return lax.bitcast_convert_type(w, _f32)


def _unpack_kv(kv):
    w = lax.bitcast_convert_type(kv, jnp.uint32)
    k = lax.bitcast_convert_type(w << 16, _f32)
    v = lax.bitcast_convert_type(w & jnp.uint32(0xFFFF0000), _f32)
    return k, v


def _qkv_body(h_ref, wq, bq, wk, bk, wv, bv,
              qp_o, kv0_o, kv1_o):
    hb = h_ref[...]
    q = jnp.dot(hb, wq[...], preferred_element_type=_f32) + bq[...]
    k = jnp.dot(hb, wk[...], preferred_element_type=_f32) + bk[...]
    v = jnp.dot(hb, wv[...], preferred_element_type=_f32) + bv[...]
    qp_o[...] = _pack_kv(q[:, :AC], q[:, AC:])
    kv0_o[...] = _pack_kv(k[:, :AC], v[:, :AC])
    kv1_o[...] = _pack_kv(k[:, AC:], v[:, AC:])


def _tc_qkv(h, wq, bq, wk, bk, wv320, bv320):
    half = jax.ShapeDtypeStruct((NP, AC), _f32)
    return pl.pallas_call(
        _qkv_body,
        grid=(NP // BN,),
        in_specs=[_rows(H, BN), _full((H, H)), _full((1, H)),
                  _full((H, H)), _full((1, H)),
                  _full((H, H)), _full((1, H))],
        out_specs=[_rows(AC, BN)] * 3,
        out_shape=[half] * 3,
    )(h, wq, bq, wk, bk, wv320, bv320)


def _matbias_body(x_ref, w, b, o_ref):
    y = jnp.dot(x_ref[...], w[...], preferred_element_type=_f32) + b[...]
    o_ref[...] = y.astype(o_ref.dtype)


def _tc_matbias(x, w, b, blk, out_dtype=_f32):
    rows = x.shape[0]
    cols = w.shape[1]
    return pl.pallas_call(
        _matbias_body,
        grid=(rows // blk,),
        in_specs=[_rows(x.shape[1], blk), _full((x.shape[1], cols)),
                  _full((1, cols))],
        out_specs=_rows(cols, blk),
        out_shape=jax.ShapeDtypeStruct((rows, cols), out_dtype),
    )(x, w, b)


def _edge_body(kvs0_ref, kvs1_ref, qps_ref, e_ref,
               ew, eb, ow, ob, f1w, f1b, f2w, f2b, g1, b1, g2, b2,
               u_o, e2_o):
    ks0, vs0 = _unpack_kv(kvs0_ref[...])
    ks1, vs1 = _unpack_kv(kvs1_ref[...])
    ks = jnp.concatenate([ks0, ks1], axis=1)
    q0, q1 = _unpack_kv(qps_ref[...])
    qd = jnp.concatenate([q0, q1], axis=1)
    eblk = e_ref[...]
    ep = _bdot(eblk, ew[...]) + eb[...]
    sarr = ks * qd * ep
    ci = lax.broadcasted_iota(jnp.int32, (H, HEADS), 0) // DK
    hi = lax.broadcasted_iota(jnp.int32, (H, HEADS), 1)
    msum = (ci == hi).astype(_f32)
    att = jnp.exp(jnp.clip(
        jnp.dot(sarr, msum, preferred_element_type=_f32), -5.0, 5.0))
    hb2 = lax.broadcasted_iota(jnp.int32, (HEADS, UW), 0)
    cb2 = lax.broadcasted_iota(jnp.int32, (HEADS, UW), 1)
    mbc = (jnp.where(cb2 < H, cb2 // DK, cb2 - H) == hb2).astype(_f32)
    vsc = jnp.concatenate(
        [vs0, vs1, jnp.ones((qps_ref.shape[0], UW - H), _f32)], axis=1)
    u_o[...] = jnp.dot(att, mbc, preferred_element_type=_f32) * vsc
    e_o = _bdot(sarr, ow[...]) + ob[...]
    e1 = _ln(eblk.astype(_f32) + e_o, g1[...], b1[...])
    ef = _bdot(jnp.maximum(_bdot(e1, f1w[...]) + f1b[...], 0.0),
               f2w[...]) + f2b[...]
    e2_o[...] = _ln(e1 + ef, g2[...], b2[...]).astype(_bf16)


def _tc_edge(kvs0, kvs1, qps, e,
             ew, eb, ow, ob, f1w, f1b, f2w, f2b, g1, b1, g2, b2):
    return pl.pallas_call(
        _edge_body,
        grid=(NE // BE,),
        in_specs=[_rows(AC, BE)] * 3 + [_rows(H, BE),
                  _full((H, H)), _full((1, H)),
                  _full((H, H)), _full((1, H)),
                  _full((H, 2 * H)), _full((1, 2 * H)),
                  _full((2 * H, H)), _full((1, H)),
                  _full((1, H)), _full((1, H)), _full((1, H)), _full((1, H))],
        out_specs=[_rows(UW, BE), _rows(H, BE)],
        out_shape=[jax.ShapeDtypeStruct((NE, UW), _f32),
                   jax.ShapeDtypeStruct((NE, H), _bf16)],
    )(kvs0, kvs1, qps, e,
      ew, eb, ow, ob, f1w, f1b, f2w, f2b, g1, b1, g2, b2)


def _node_body(s_ref, h_ref, ow, ob, f1w, f1b, f2w, f2b, g1, b1, g2, b2, h2_o):
    sblk = s_ref[...]
    wv = sblk[:, :H]
    ri = lax.broadcasted_iota(jnp.int32, (ZW, H), 0)
    ci = lax.broadcasted_iota(jnp.int32, (ZW, H), 1)
    rh = jnp.where(ri >= H + AC, ri - H - AC, ri - H)
    mz = ((ri >= H) & (rh < HEADS) & (ci // DK == rh)).astype(_f32)
    zb = jnp.dot(sblk, mz, preferred_element_type=_f32) + 1e-6
    hat = wv / zb
    h_o = _bdot(hat, ow[...]) + ob[...]
    h1 = _ln(h_ref[...] + h_o, g1[...], b1[...])
    hf = _bdot(jnp.maximum(_bdot(h1, f1w[...]) + f1b[...], 0.0),
               f2w[...]) + f2b[...]
    h2_o[...] = _ln(h1 + hf, g2[...], b2[...])


def _tc_node(sacc, h, ow, ob, f1w, f1b, f2w, f2b, g1, b1, g2, b2):
    return pl.pallas_call(
        _node_body,
        grid=(NP // BN,),
        in_specs=[_rows(ZW, BN), _rows(H, BN),
                  _full((H, H)), _full((1, H)),
                  _full((H, 2 * H)), _full((1, 2 * H)),
                  _full((2 * H, H)), _full((1, H)),
                  _full((1, H)), _full((1, H)), _full((1, H)), _full((1, H))],
        out_specs=_rows(H, BN),
        out_shape=jax.ShapeDtypeStruct((NP, H), _f32),
    )(sacc, h, ow, ob, f1w, f1b, f2w, f2b, g1, b1, g2, b2)


def _sc_gather3(kv0, kv1, qp, src, dst):
    mesh = plsc.VectorSubcoreMesh(core_axis_name="c", subcore_axis_name="s")
    ch = NE // 16
    gb = 176
    np2 = 28
    tb = ch - np2 * 2 * gb
    tr = NP // 16

    @functools.partial(
        pl.kernel,
        out_type=[jax.ShapeDtypeStruct((NE, AC), _f32)] * 3,
        mesh=mesh,
        scratch_types=[
            pltpu.VMEM((gb,), jnp.int32),
            pltpu.VMEM((gb,), jnp.int32),
            pltpu.VMEM((gb, AC), _f32),
            pltpu.VMEM((gb, AC), _f32),
            pltpu.VMEM_SHARED((NP, AC), _f32),
            pltpu.SemaphoreType.DMA,
            pltpu.SemaphoreType.DMA,
        ],
    )
    def kfn(kv0_hbm, kv1_hbm, qp_hbm,
            src_hbm, dst_hbm,
            kvs0_hbm, kvs1_hbm, qps_hbm,
            idx0, idx1, rows0, rows1, spm, semg, semw):
        c = lax.axis_index("c")
        s = lax.axis_index("s")

        def phase(tab_hbm, idx_hbm, out_hbm, base0, ch_t, np_t, tails):
            pltpu.sync_copy(tab_hbm.at[pl.ds(s * tr, tr)],
                            spm.at[pl.ds(s * tr, tr)])
            plsc.subcore_barrier()

            def body(j2, carry):
                base = base0 + s * ch_t + j2 * (2 * gb)

                @pl.when(j2 > 0)
                def _():
                    pltpu.make_async_copy(
                        rows0, out_hbm.at[pl.ds(0, gb)], semw).wait()
                    pltpu.make_async_copy(
                        rows1, out_hbm.at[pl.ds(0, gb)], semw).wait()

                pltpu.sync_copy(idx_hbm.at[pl.ds(base, gb)], idx0)
                pltpu.sync_copy(idx_hbm.at[pl.ds(base + gb, gb)], idx1)
                pltpu.async_copy(spm.at[idx0], rows0, semg).wait()
                pltpu.async_copy(rows0, out_hbm.at[pl.ds(base, gb)], semw)
                pltpu.async_copy(spm.at[idx1], rows1, semg).wait()
                pltpu.async_copy(rows1, out_hbm.at[pl.ds(base + gb, gb)],
                                 semw)
                return carry

            lax.fori_loop(0, np_t, body, 0)
            pltpu.make_async_copy(rows0, out_hbm.at[pl.ds(0, gb)], semw).wait()
            pltpu.make_async_copy(rows1, out_hbm.at[pl.ds(0, gb)], semw).wait()
            tbase = base0 + s * ch_t + np_t * 2 * gb
            for t in tails:
                pltpu.sync_copy(idx_hbm.at[pl.ds(tbase, t)],
                                idx0.at[pl.ds(0, t)])
                pltpu.async_copy(spm.at[idx0.at[pl.ds(0, t)]],
                                 rows0.at[pl.ds(0, t)], semg).wait()
                pltpu.sync_copy(rows0.at[pl.ds(0, t)],
                                out_hbm.at[pl.ds(tbase, t)])
                tbase = tbase + t
            plsc.subcore_barrier()

        @pl.when(c == 0)
        def _():
            phase(kv0_hbm, src_hbm, kvs0_hbm, 0, ch, np2, [tb])
            phase(qp_hbm, dst_hbm, qps_hbm, 0, NE // 32, 14, [72])

        @pl.when(c == 1)
        def _():
            phase(kv1_hbm, src_hbm, kvs1_hbm, 0, ch, np2, [tb])
            phase(qp_hbm, dst_hbm, qps_hbm, NE // 2, NE // 32, 14, [72])

    return kfn(kv0, kv1, qp, src, dst)


def _sc_scatter(u, dst, zinit):
    mesh = plsc.VectorSubcoreMesh(core_axis_name="c", subcore_axis_name="s")
    sb = 160
    ch1 = 9920
    np1 = ch1 // (2 * sb)
    ch2 = 4960
    n2 = ch2 // sb
    zr = NP // 16

    @functools.partial(
        pl.kernel,
        out_type=jax.ShapeDtypeStruct((NP, ZW), _f32),
        mesh=mesh,
        scratch_types=[
            pltpu.VMEM((sb,), jnp.int32),
            pltpu.VMEM((sb,), jnp.int32),
            pltpu.VMEM((sb, AC), _f32),
            pltpu.VMEM((sb, AC), _f32),
            pltpu.VMEM_SHARED((NP, AC), _f32),
            pltpu.SemaphoreType.DMA,
            pltpu.SemaphoreType.DMA,
            pltpu.SemaphoreType.DMA,
            pltpu.SemaphoreType.DMA,
        ],
    )
    def kfn(u_hbm, dst_hbm, z_hbm, out_hbm, idx0, idx1, st0, st1,
            acc, semi0, semu0, semi1, semu1):
        c = lax.axis_index("c")
        s = lax.axis_index("s")
        pltpu.sync_copy(z_hbm, acc.at[pl.ds(s * zr, zr)])
        plsc.subcore_barrier()

        def load(base, idxv, stv, col0, semi, semu):
            pltpu.async_copy(dst_hbm.at[pl.ds(base, sb)], idxv, semi)
            pltpu.async_copy(u_hbm.at[pl.ds(base, sb), pl.ds(col0, AC)],
                             stv, semu)

        def drain(idxv, stv, col0, semi, semu):
            pltpu.make_async_copy(dst_hbm.at[pl.ds(0, sb)], idxv, semi).wait()
            pltpu.make_async_copy(u_hbm.at[pl.ds(0, sb), pl.ds(col0, AC)],
                                  stv, semu).wait()

        def block(base, col0):
            pltpu.sync_copy(dst_hbm.at[pl.ds(base, sb)], idx0)
            pltpu.sync_copy(u_hbm.at[pl.ds(base, sb), pl.ds(col0, AC)], st0)
            pltpu.sync_copy(st0, acc.at[idx0], add=True)

        def sweep1(col0):
            base0 = s * ch1
            load(base0, idx0, st0, col0, semi0, semu0)

            def body(j2, carry):
                base = base0 + j2 * (2 * sb)
                load(base + sb, idx1, st1, col0, semi1, semu1)
                drain(idx0, st0, col0, semi0, semu0)
                pltpu.sync_copy(st0, acc.at[idx0], add=True)

                @pl.when(j2 < np1 - 1)
                def _():
                    load(base + 2 * sb, idx0, st0, col0, semi0, semu0)

                drain(idx1, st1, col0, semi1, semu1)
                pltpu.sync_copy(st1, acc.at[idx1], add=True)
                return carry

            lax.fori_loop(0, np1, body, 0)

            @pl.when(s < 4)
            def _():
                block(16 * ch1 + s * 2 * sb, col0)
                block(16 * ch1 + s * 2 * sb + sb, col0)

        def sweep2(col0):
            base0 = c * (NE // 2) + s * ch2

            def body(j, carry):
                block(base0 + j * sb, col0)
                return carry

            lax.fori_loop(0, n2, body, 0)

            @pl.when(s < 4)
            def _():
                block(c * (NE // 2) + 16 * ch2 + s * sb, col0)

        def copyout(col0):
            pltpu.sync_copy(acc.at[pl.ds(s * zr, zr)],
                            out_hbm.at[pl.ds(s * zr, zr), pl.ds(col0, AC)])

        @pl.when(c == 0)
        def _():
            sweep1(0)

        @pl.when(c == 1)
        def _():
            sweep1(AC)

        plsc.subcore_barrier()

        @pl.when(c == 0)
        def _():
            copyout(0)

        @pl.when(c == 1)
        def _():
            copyout(AC)

        pltpu.sync_copy(z_hbm, acc.at[pl.ds(s * zr, zr)])
        plsc.subcore_barrier()

        sweep2(2 * AC)
        plsc.subcore_barrier()

        @pl.when(c == 0)
        def _():
            copyout(2 * AC)

        @pl.when(c == 1)
        def _():
            copyout(3 * AC)

    return kfn(u, dst, zinit)


def kernel(g, h, e, params):
    src = g[0].astype(jnp.int32)
    dst = g[1].astype(jnp.int32)
    h = jnp.pad(h, ((0, NP - h.shape[0]), (0, 0)))
    zinit = jnp.zeros((NP // 16, AC), _f32)
    scale = np.float32(1.0 / np.sqrt(DK))

    def r(b):
        return b.reshape(1, -1)

    for p in params["layers"]:
        qp, kv0, kv1 = _tc_qkv(h, p["Q"]["W"], r(p["Q"]["b"]),
                               p["K"]["W"], r(p["K"]["b"]),
                               p["V"]["W"], r(p["V"]["b"]))
        kvs0, kvs1, qps = _sc_gather3(kv0, kv1, qp, src, dst)
        u, e = _tc_edge(kvs0, kvs1, qps, e,
                        (p["E"]["W"] * scale).astype(_bf16),
                        r(p["E"]["b"]) * scale,
                        p["Oe"]["W"].astype(_bf16), r(p["Oe"]["b"]),
                        p["Fe1"]["W"].astype(_bf16), r(p["Fe1"]["b"]),
                        p["Fe2"]["W"].astype(_bf16), r(p["Fe2"]["b"]),
                        r(p["ln1e_g"]), r(p["ln1e_b"]),
                        r(p["ln2e_g"]), r(p["ln2e_b"]))
        sacc = _sc_scatter(u, dst, zinit)
        h = _tc_node(sacc, h,
                     p["Oh"]["W"].astype(_bf16), r(p["Oh"]["b"]),
                     p["Fh1"]["W"].astype(_bf16), r(p["Fh1"]["b"]),
                     p["Fh2"]["W"].astype(_bf16), r(p["Fh2"]["b"]),
                     r(p["ln1h_g"]), r(p["ln1h_b"]),
                     r(p["ln2h_g"]), r(p["ln2h_b"]))

    cw = jnp.pad(params["cls"]["W"], ((0, 0), (0, OUTP - 40)))
    cb = jnp.pad(params["cls"]["b"], ((0, OUTP - 40),)).reshape(1, OUTP)
    logits = _tc_matbias(h, cw, cb, BN)
    return logits[:N_REAL, :40]


# speedup vs baseline: 1.0595x; 1.0485x over previous
import functools

import jax
import jax.numpy as jnp
import numpy as np
from jax import lax
from jax.experimental import pallas as pl
from jax.experimental.pallas import tpu as pltpu
from jax.experimental.pallas import tpu_sc as plsc

H = 256
HEADS = 8
DK = 32
N_REAL = 10000
NP = 10240
NE = 160000
UW = 384
ZW = 512
AC = 128
OUTP = 128

BN = 512
BE = 800
GB = 200
SB = 200

_f32 = jnp.float32
_bf16 = jnp.bfloat16


def _bdot(a, w):
    return jnp.dot(a.astype(_bf16), w, preferred_element_type=_f32)


def _full(shape):
    return pl.BlockSpec(shape, lambda i: (0,) * len(shape))


def _rows(width, blk):
    return pl.BlockSpec((blk, width), lambda i: (i, 0))


def _ln(x, g, b):
    mu = jnp.mean(x, axis=-1, keepdims=True)
    var = jnp.mean((x - mu) ** 2, axis=-1, keepdims=True)
    return (x - mu) * lax.rsqrt(var + 1e-5) * g + b


def _pack_kv(k, v):
    ku = lax.bitcast_convert_type(k.astype(_bf16), jnp.uint16)
    vu = lax.bitcast_convert_type(v.astype(_bf16), jnp.uint16)
    w = ku.astype(jnp.uint32) | (vu.astype(jnp.uint32) << 16)
    return lax.bitcast_convert_type(w, _f32)


def _unpack_kv(kv):
    w = lax.bitcast_convert_type(kv, jnp.uint32)
    k = lax.bitcast_convert_type(w << 16, _f32)
    v = lax.bitcast_convert_type(w & jnp.uint32(0xFFFF0000), _f32)
    return k, v


def _qkv_body(h_ref, wq, bq, wk, bk, wv, bv,
              qp_o, kv0_o, kv1_o):
    hb = h_ref[...]
    q = jnp.dot(hb, wq[...], preferred_element_type=_f32) + bq[...]
    k = jnp.dot(hb, wk[...], preferred_element_type=_f32) + bk[...]
    v = jnp.dot(hb, wv[...], preferred_element_type=_f32) + bv[...]
    qp_o[...] = _pack_kv(q[:, :AC], q[:, AC:])
    kv0_o[...] = _pack_kv(k[:, :AC], v[:, :AC])
    kv1_o[...] = _pack_kv(k[:, AC:], v[:, AC:])


def _tc_qkv(h, wq, bq, wk, bk, wv320, bv320):
    half = jax.ShapeDtypeStruct((NP, AC), _f32)
    return pl.pallas_call(
        _qkv_body,
        grid=(NP // BN,),
        in_specs=[_rows(H, BN), _full((H, H)), _full((1, H)),
                  _full((H, H)), _full((1, H)),
                  _full((H, H)), _full((1, H))],
        out_specs=[_rows(AC, BN)] * 3,
        out_shape=[half] * 3,
    )(h, wq, bq, wk, bk, wv320, bv320)


def _matbias_body(x_ref, w, b, o_ref):
    y = jnp.dot(x_ref[...], w[...], preferred_element_type=_f32) + b[...]
    o_ref[...] = y.astype(o_ref.dtype)


def _tc_matbias(x, w, b, blk, out_dtype=_f32):
    rows = x.shape[0]
    cols = w.shape[1]
    return pl.pallas_call(
        _matbias_body,
        grid=(rows // blk,),
        in_specs=[_rows(x.shape[1], blk), _full((x.shape[1], cols)),
                  _full((1, cols))],
        out_specs=_rows(cols, blk),
        out_shape=jax.ShapeDtypeStruct((rows, cols), out_dtype),
    )(x, w, b)


def _edge_body(kvs0_ref, kvs1_ref, qps_ref, e_ref,
               ew, eb, ow, ob, f1w, f1b, f2w, f2b, g1, b1, g2, b2,
               u_o, e2_o):
    ks0, vs0 = _unpack_kv(kvs0_ref[...])
    ks1, vs1 = _unpack_kv(kvs1_ref[...])
    ks = jnp.concatenate([ks0, ks1], axis=1)
    q0, q1 = _unpack_kv(qps_ref[...])
    qd = jnp.concatenate([q0, q1], axis=1)
    eblk = e_ref[...]
    ep = _bdot(eblk, ew[...]) + eb[...]
    sarr = ks * qd * ep
    ci = lax.broadcasted_iota(jnp.int32, (H, HEADS), 0) // DK
    hi = lax.broadcasted_iota(jnp.int32, (H, HEADS), 1)
    msum = (ci == hi).astype(_f32)
    att = jnp.exp(jnp.clip(
        jnp.dot(sarr, msum, preferred_element_type=_f32), -5.0, 5.0))
    hb2 = lax.broadcasted_iota(jnp.int32, (HEADS, UW), 0)
    cb2 = lax.broadcasted_iota(jnp.int32, (HEADS, UW), 1)
    mbc = (jnp.where(cb2 < H, cb2 // DK, cb2 - H) == hb2).astype(_f32)
    vsc = jnp.concatenate(
        [vs0, vs1, jnp.ones((qps_ref.shape[0], UW - H), _f32)], axis=1)
    u_o[...] = jnp.dot(att, mbc, preferred_element_type=_f32) * vsc
    e_o = _bdot(sarr, ow[...]) + ob[...]
    e1 = _ln(eblk.astype(_f32) + e_o, g1[...], b1[...])
    ef = _bdot(jnp.maximum(_bdot(e1, f1w[...]) + f1b[...], 0.0),
               f2w[...]) + f2b[...]
    e2_o[...] = _ln(e1 + ef, g2[...], b2[...]).astype(_bf16)


def _tc_edge(kvs0, kvs1, qps, e,
             ew, eb, ow, ob, f1w, f1b, f2w, f2b, g1, b1, g2, b2):
    return pl.pallas_call(
        _edge_body,
        grid=(NE // BE,),
        in_specs=[_rows(AC, BE)] * 3 + [_rows(H, BE),
                  _full((H, H)), _full((1, H)),
                  _full((H, H)), _full((1, H)),
                  _full((H, 2 * H)), _full((1, 2 * H)),
                  _full((2 * H, H)), _full((1, H)),
                  _full((1, H)), _full((1, H)), _full((1, H)), _full((1, H))],
        out_specs=[_rows(UW, BE), _rows(H, BE)],
        out_shape=[jax.ShapeDtypeStruct((NE, UW), _f32),
                   jax.ShapeDtypeStruct((NE, H), _bf16)],
    )(kvs0, kvs1, qps, e,
      ew, eb, ow, ob, f1w, f1b, f2w, f2b, g1, b1, g2, b2)


def _node_body(s_ref, h_ref, ow, ob, f1w, f1b, f2w, f2b, g1, b1, g2, b2, h2_o):
    sblk = s_ref[...]
    wv = sblk[:, :H]
    ri = lax.broadcasted_iota(jnp.int32, (ZW, H), 0)
    ci = lax.broadcasted_iota(jnp.int32, (ZW, H), 1)
    rh = jnp.where(ri >= H + AC, ri - H - AC, ri - H)
    mz = ((ri >= H) & (rh < HEADS) & (ci // DK == rh)).astype(_f32)
    zb = jnp.dot(sblk, mz, preferred_element_type=_f32) + 1e-6
    hat = wv / zb
    h_o = _bdot(hat, ow[...]) + ob[...]
    h1 = _ln(h_ref[...] + h_o, g1[...], b1[...])
    hf = _bdot(jnp.maximum(_bdot(h1, f1w[...]) + f1b[...], 0.0),
               f2w[...]) + f2b[...]
    h2_o[...] = _ln(h1 + hf, g2[...], b2[...])


def _tc_node(sacc, h, ow, ob, f1w, f1b, f2w, f2b, g1, b1, g2, b2):
    return pl.pallas_call(
        _node_body,
        grid=(NP // BN,),
        in_specs=[_rows(ZW, BN), _rows(H, BN),
                  _full((H, H)), _full((1, H)),
                  _full((H, 2 * H)), _full((1, 2 * H)),
                  _full((2 * H, H)), _full((1, H)),
                  _full((1, H)), _full((1, H)), _full((1, H)), _full((1, H))],
        out_specs=_rows(H, BN),
        out_shape=jax.ShapeDtypeStruct((NP, H), _f32),
    )(sacc, h, ow, ob, f1w, f1b, f2w, f2b, g1, b1, g2, b2)


def _sc_gather3(kv0, kv1, qp, src, dst):
    mesh = plsc.VectorSubcoreMesh(core_axis_name="c", subcore_axis_name="s")
    ch = NE // 16
    gb = 176
    np2 = 28
    tb = ch - np2 * 2 * gb
    tr = NP // 16

    @functools.partial(
        pl.kernel,
        out_type=[jax.ShapeDtypeStruct((NE, AC), _f32)] * 3,
        mesh=mesh,
        scratch_types=[
            pltpu.VMEM((gb,), jnp.int32),
            pltpu.VMEM((gb,), jnp.int32),
            pltpu.VMEM((gb, AC), _f32),
            pltpu.VMEM((gb, AC), _f32),
            pltpu.VMEM_SHARED((NP, AC), _f32),
            pltpu.SemaphoreType.DMA,
            pltpu.SemaphoreType.DMA,
            pltpu.SemaphoreType.DMA,
            pltpu.SemaphoreType.DMA,
        ],
    )
    def kfn(kv0_hbm, kv1_hbm, qp_hbm,
            src_hbm, dst_hbm,
            kvs0_hbm, kvs1_hbm, qps_hbm,
            idx0, idx1, rows0, rows1, spm, semg, semw, semi0, semi1):
        c = lax.axis_index("c")
        s = lax.axis_index("s")

        def phase(tab_hbm, idx_hbm, out_hbm, base0, ch_t, np_t, tails):
            pltpu.sync_copy(tab_hbm.at[pl.ds(s * tr, tr)],
                            spm.at[pl.ds(s * tr, tr)])
            plsc.subcore_barrier()

            def body(j2, carry):
                base = base0 + s * ch_t + j2 * (2 * gb)

                @pl.when(j2 > 0)
                def _():
                    pltpu.make_async_copy(
                        rows0, out_hbm.at[pl.ds(0, gb)], semw).wait()
                    pltpu.make_async_copy(
                        rows1, out_hbm.at[pl.ds(0, gb)], semw).wait()
                    pltpu.make_async_copy(
                        idx_hbm.at[pl.ds(0, gb)], idx0, semi0).wait()
                    pltpu.make_async_copy(
                        idx_hbm.at[pl.ds(0, gb)], idx1, semi1).wait()

                @pl.when(j2 == 0)
                def _():
                    pltpu.sync_copy(idx_hbm.at[pl.ds(base, gb)], idx0)
                    pltpu.sync_copy(idx_hbm.at[pl.ds(base + gb, gb)], idx1)

                pltpu.async_copy(spm.at[idx0], rows0, semg).wait()
                pltpu.async_copy(rows0, out_hbm.at[pl.ds(base, gb)], semw)

                @pl.when(j2 < np_t - 1)
                def _():
                    pltpu.async_copy(
                        idx_hbm.at[pl.ds(base + 2 * gb, gb)], idx0, semi0)

                pltpu.async_copy(spm.at[idx1], rows1, semg).wait()
                pltpu.async_copy(rows1, out_hbm.at[pl.ds(base + gb, gb)],
                                 semw)

                @pl.when(j2 < np_t - 1)
                def _():
                    pltpu.async_copy(
                        idx_hbm.at[pl.ds(base + 3 * gb, gb)], idx1, semi1)

                return carry

            lax.fori_loop(0, np_t, body, 0)
            pltpu.make_async_copy(rows0, out_hbm.at[pl.ds(0, gb)], semw).wait()
            pltpu.make_async_copy(rows1, out_hbm.at[pl.ds(0, gb)], semw).wait()
            tbase = base0 + s * ch_t + np_t * 2 * gb
            for t in tails:
                pltpu.sync_copy(idx_hbm.at[pl.ds(tbase, t)],
                                idx0.at[pl.ds(0, t)])
                pltpu.async_copy(spm.at[idx0.at[pl.ds(0, t)]],
                                 rows0.at[pl.ds(0, t)], semg).wait()
                pltpu.sync_copy(rows0.at[pl.ds(0, t)],
                                out_hbm.at[pl.ds(tbase, t)])
                tbase = tbase + t
            plsc.subcore_barrier()

        @pl.when(c == 0)
        def _():
            phase(kv0_hbm, src_hbm, kvs0_hbm, 0, ch, np2, [tb])
            phase(qp_hbm, dst_hbm, qps_hbm, 0, NE // 32, 14, [72])

        @pl.when(c == 1)
        def _():
            phase(kv1_hbm, src_hbm, kvs1_hbm, 0, ch, np2, [tb])
            phase(qp_hbm, dst_hbm, qps_hbm, NE // 2, NE // 32, 14, [72])

    return kfn(kv0, kv1, qp, src, dst)


def _sc_scatter(u, dst, zinit):
    mesh = plsc.VectorSubcoreMesh(core_axis_name="c", subcore_axis_name="s")
    sb = 160
    ch1 = 9920
    np1 = ch1 // (2 * sb)
    ch2 = 4960
    n2 = ch2 // sb
    zr = NP // 16

    @functools.partial(
        pl.kernel,
        out_type=jax.ShapeDtypeStruct((NP, ZW), _f32),
        mesh=mesh,
        scratch_types=[
            pltpu.VMEM((sb,), jnp.int32),
            pltpu.VMEM((sb,), jnp.int32),
            pltpu.VMEM((sb, AC), _f32),
            pltpu.VMEM((sb, AC), _f32),
            pltpu.VMEM_SHARED((NP, AC), _f32),
            pltpu.SemaphoreType.DMA,
            pltpu.SemaphoreType.DMA,
            pltpu.SemaphoreType.DMA,
            pltpu.SemaphoreType.DMA,
        ],
    )
    def kfn(u_hbm, dst_hbm, z_hbm, out_hbm, idx0, idx1, st0, st1,
            acc, semi0, semu0, semi1, semu1):
        c = lax.axis_index("c")
        s = lax.axis_index("s")
        pltpu.sync_copy(z_hbm, acc.at[pl.ds(s * zr, zr)])
        plsc.subcore_barrier()

        def load(base, idxv, stv, col0, semi, semu):
            pltpu.async_copy(dst_hbm.at[pl.ds(base, sb)], idxv, semi)
            pltpu.async_copy(u_hbm.at[pl.ds(base, sb), pl.ds(col0, AC)],
                             stv, semu)

        def drain(idxv, stv, col0, semi, semu):
            pltpu.make_async_copy(dst_hbm.at[pl.ds(0, sb)], idxv, semi).wait()
            pltpu.make_async_copy(u_hbm.at[pl.ds(0, sb), pl.ds(col0, AC)],
                                  stv, semu).wait()

        def block(base, col0):
            pltpu.sync_copy(dst_hbm.at[pl.ds(base, sb)], idx0)
            pltpu.sync_copy(u_hbm.at[pl.ds(base, sb), pl.ds(col0, AC)], st0)
            pltpu.sync_copy(st0, acc.at[idx0], add=True)

        def sweep1(col0):
            base0 = s * ch1
            load(base0, idx0, st0, col0, semi0, semu0)

            def body(j2, carry):
                base = base0 + j2 * (2 * sb)
                load(base + sb, idx1, st1, col0, semi1, semu1)
                drain(idx0, st0, col0, semi0, semu0)
                pltpu.sync_copy(st0, acc.at[idx0], add=True)

                @pl.when(j2 < np1 - 1)
                def _():
                    load(base + 2 * sb, idx0, st0, col0, semi0, semu0)

                drain(idx1, st1, col0, semi1, semu1)
                pltpu.sync_copy(st1, acc.at[idx1], add=True)
                return carry

            lax.fori_loop(0, np1, body, 0)

            @pl.when(s < 4)
            def _():
                block(16 * ch1 + s * 2 * sb, col0)
                block(16 * ch1 + s * 2 * sb + sb, col0)

        def sweep2(col0):
            base0 = c * (NE // 2) + s * ch2

            def body(j, carry):
                block(base0 + j * sb, col0)
                return carry

            lax.fori_loop(0, n2, body, 0)

            @pl.when(s < 4)
            def _():
                block(c * (NE // 2) + 16 * ch2 + s * sb, col0)

        def copyout(col0):
            pltpu.sync_copy(acc.at[pl.ds(s * zr, zr)],
                            out_hbm.at[pl.ds(s * zr, zr), pl.ds(col0, AC)])

        @pl.when(c == 0)
        def _():
            sweep1(0)

        @pl.when(c == 1)
        def _():
            sweep1(AC)

        plsc.subcore_barrier()

        @pl.when(c == 0)
        def _():
            copyout(0)

        @pl.when(c == 1)
        def _():
            copyout(AC)

        pltpu.sync_copy(z_hbm, acc.at[pl.ds(s * zr, zr)])
        plsc.subcore_barrier()

        sweep2(2 * AC)
        plsc.subcore_barrier()

        @pl.when(c == 0)
        def _():
            copyout(2 * AC)

        @pl.when(c == 1)
        def _():
            copyout(3 * AC)

    return kfn(u, dst, zinit)


def kernel(g, h, e, params):
    src = g[0].astype(jnp.int32)
    dst = g[1].astype(jnp.int32)
    h = jnp.pad(h, ((0, NP - h.shape[0]), (0, 0)))
    zinit = jnp.zeros((NP // 16, AC), _f32)
    scale = np.float32(1.0 / np.sqrt(DK))

    def r(b):
        return b.reshape(1, -1)

    for p in params["layers"]:
        qp, kv0, kv1 = _tc_qkv(h, p["Q"]["W"], r(p["Q"]["b"]),
                               p["K"]["W"], r(p["K"]["b"]),
                               p["V"]["W"], r(p["V"]["b"]))
        kvs0, kvs1, qps = _sc_gather3(kv0, kv1, qp, src, dst)
        u, e = _tc_edge(kvs0, kvs1, qps, e,
                        (p["E"]["W"] * scale).astype(_bf16),
                        r(p["E"]["b"]) * scale,
                        p["Oe"]["W"].astype(_bf16), r(p["Oe"]["b"]),
                        p["Fe1"]["W"].astype(_bf16), r(p["Fe1"]["b"]),
                        p["Fe2"]["W"].astype(_bf16), r(p["Fe2"]["b"]),
                        r(p["ln1e_g"]), r(p["ln1e_b"]),
                        r(p["ln2e_g"]), r(p["ln2e_b"]))
        sacc = _sc_scatter(u, dst, zinit)
        h = _tc_node(sacc, h,
                     p["Oh"]["W"].astype(_bf16), r(p["Oh"]["b"]),
                     p["Fh1"]["W"].astype(_bf16), r(p["Fh1"]["b"]),
                     p["Fh2"]["W"].astype(_bf16), r(p["Fh2"]["b"]),
                     r(p["ln1h_g"]), r(p["ln1h_b"]),
                     r(p["ln2h_g"]), r(p["ln2h_b"]))

    cw = jnp.pad(params["cls"]["W"], ((0, 0), (0, OUTP - 40)))
    cb = jnp.pad(params["cls"]["b"], ((0, OUTP - 40),)).reshape(1, OUTP)
    logits = _tc_matbias(h, cw, cb, BN)
    return logits[:N_REAL, :40]
